# Initial kernel scaffold; baseline (speedup 1.0000x reference)
import functools

import jax
import jax.numpy as jnp
from jax import lax
from jax.experimental import pallas as pl
from jax.experimental.pallas import tpu as pltpu
from jax.experimental.pallas import tpu_sc as plsc

N = 10000
E = 320000
D = 128
HALF = 64
W = 80
K = 80
NS = 16
NC = 2
NCHUNK_TOT = E // K
NCHUNK = NCHUNK_TOT // NS
RPT = N // NS


def _make_agg():
    mesh = plsc.VectorSubcoreMesh(core_axis_name="c", subcore_axis_name="s")

    @functools.partial(
        pl.kernel,
        mesh=mesh,
        out_type=jax.ShapeDtypeStruct((2 * N, W), jnp.float32),
        scratch_types=[
            pltpu.VMEM_SHARED((N, W), jnp.float32),
            pltpu.VMEM((K,), jnp.int32),
            pltpu.VMEM((K,), jnp.int32),
            pltpu.VMEM((K, W), jnp.float32),
            pltpu.SemaphoreType.DMA,
        ],
    )
    def agg(xcat, srcall, dst2d, zeros, out, acc, srcv, dstv, rows, sem):
        c = lax.axis_index("c")
        s = lax.axis_index("s")
        pltpu.sync_copy(zeros.at[pl.ds(s * RPT, RPT)],
                        acc.at[pl.ds(s * RPT, RPT)])
        plsc.subcore_barrier()

        base = s * NCHUNK

        def body(i, carry):
            pltpu.sync_copy(srcall.at[c, base + i], srcv)
            pltpu.sync_copy(dst2d.at[base + i], dstv)
            pltpu.async_copy(xcat.at[srcv], rows, sem).wait()
            pltpu.sync_copy(rows, acc.at[dstv], add=True)
            return carry

        lax.fori_loop(0, NCHUNK, body, 0)
        plsc.subcore_barrier()
        pltpu.sync_copy(acc.at[pl.ds(s * RPT, RPT)],
                        out.at[pl.ds(c * N + s * RPT, RPT)])

    return agg


_agg = _make_agg()


BN = 400


def _tc1_body(p0, p1, xr, wl, wr, br, o):
    cnt = jnp.maximum(p1[:, HALF:HALF + 1], 1.0)
    ml = p0[:, :HALF] / cnt
    mh = p1[:, :HALF] / cnt
    z = (jnp.dot(ml, wl[:HALF, :], preferred_element_type=jnp.float32)
         + jnp.dot(mh, wl[HALF:, :], preferred_element_type=jnp.float32)
         + jnp.dot(xr[...], wr[...], preferred_element_type=jnp.float32)
         + br[...])
    h = jnp.maximum(z, 0.0)
    pad = jnp.zeros((BN, W - HALF), jnp.float32)
    o[0] = jnp.concatenate([h[:, :HALF], pad], axis=1)
    o[1] = jnp.concatenate([h[:, HALF:], pad], axis=1)


def _tc1(agg1, x, Wl1, Wr1, b1):
    nb = N // BN
    return pl.pallas_call(
        _tc1_body,
        grid=(nb,),
        in_specs=[
            pl.BlockSpec((BN, W), lambda i: (i, 0)),
            pl.BlockSpec((BN, W), lambda i: (nb + i, 0)),
            pl.BlockSpec((BN, D), lambda i: (i, 0)),
            pl.BlockSpec((D, D), lambda i: (0, 0)),
            pl.BlockSpec((D, D), lambda i: (0, 0)),
            pl.BlockSpec((1, D), lambda i: (0, 0)),
        ],
        out_specs=pl.BlockSpec((2, BN, W), lambda i: (0, i, 0)),
        out_shape=jax.ShapeDtypeStruct((2, N, W), jnp.float32),
    )(agg1, agg1, x, Wl1, Wr1, b1)


def _tc2_body(a0, a1, c1, h2, wl, wr, br, o):
    cnt = jnp.maximum(c1[:, HALF:HALF + 1], 1.0)
    ml = a0[:, :HALF] / cnt
    mh = a1[:, :HALF] / cnt
    h0 = h2[0, :, :HALF]
    h1 = h2[1, :, :HALF]
    z = (jnp.dot(ml, wl[:HALF, :], preferred_element_type=jnp.float32)
         + jnp.dot(mh, wl[HALF:, :], preferred_element_type=jnp.float32)
         + jnp.dot(h0, wr[:HALF, :], preferred_element_type=jnp.float32)
         + jnp.dot(h1, wr[HALF:, :], preferred_element_type=jnp.float32)
         + br[...])
    o[...] = z


def _tc2(agg2, agg1, h2, Wl2, Wr2, b2):
    nb = N // BN
    return pl.pallas_call(
        _tc2_body,
        grid=(nb,),
        in_specs=[
            pl.BlockSpec((BN, W), lambda i: (i, 0)),
            pl.BlockSpec((BN, W), lambda i: (nb + i, 0)),
            pl.BlockSpec((BN, W), lambda i: (nb + i, 0)),
            pl.BlockSpec((2, BN, W), lambda i: (0, i, 0)),
            pl.BlockSpec((D, D), lambda i: (0, 0)),
            pl.BlockSpec((D, D), lambda i: (0, 0)),
            pl.BlockSpec((1, D), lambda i: (0, 0)),
        ],
        out_specs=pl.BlockSpec((BN, D), lambda i: (i, 0)),
        out_shape=jax.ShapeDtypeStruct((N, D), jnp.float32),
    )(agg2, agg2, agg1, h2, Wl2, Wr2, b2)


def kernel(x, edge_index, Wl1, Wr1, b1, Wl2, Wr2, b2):
    src = edge_index[0].astype(jnp.int32).reshape(NCHUNK_TOT, K)
    dst2d = edge_index[1].astype(jnp.int32).reshape(NCHUNK_TOT, K)
    srcall = jnp.stack([src, src + N])
    part0 = jnp.pad(x[:, :HALF], ((0, 0), (0, W - HALF)))
    part1 = jnp.concatenate(
        [x[:, HALF:], jnp.ones((N, 1), jnp.float32),
         jnp.zeros((N, W - HALF - 1), jnp.float32)], axis=1)
    xcat = jnp.concatenate([part0, part1], axis=0)
    zeros = jnp.zeros((N, W), jnp.float32)

    agg1 = _agg(xcat, srcall, dst2d, zeros)
    h2 = _tc1(agg1, x, Wl1, Wr1, b1.reshape(1, D))
    hcat = h2.reshape(2 * N, W)
    agg2 = _agg(hcat, srcall, dst2d, zeros)
    return _tc2(agg2, agg1, h2, Wl2, Wr2, b2.reshape(1, D))


# speedup vs baseline: 3.2145x; 3.2145x over previous
import functools

import jax
import jax.numpy as jnp
from jax import lax
from jax.experimental import pallas as pl
from jax.experimental.pallas import tpu as pltpu
from jax.experimental.pallas import tpu_sc as plsc

N = 10000
E = 320000
D = 128
HALF = 64
W = 80
K = 80
NS = 16
NC = 2
NCHUNK_TOT = E // K
NCHUNK = NCHUNK_TOT // NS
RPT = N // NS


def _make_agg():
    mesh = plsc.VectorSubcoreMesh(core_axis_name="c", subcore_axis_name="s")

    @functools.partial(
        pl.kernel,
        mesh=mesh,
        compiler_params=pltpu.CompilerParams(use_tc_tiling_on_sc=False),
        out_type=jax.ShapeDtypeStruct((2 * N, W), jnp.float32),
        scratch_types=[
            pltpu.VMEM_SHARED((N, W), jnp.float32),
            pltpu.VMEM((K,), jnp.int32),
            pltpu.VMEM((K,), jnp.int32),
            pltpu.VMEM((K, W), jnp.float32),
            pltpu.SemaphoreType.DMA,
        ],
    )
    def agg(xcat, srcall, dst2d, zeros, out, acc, srcv, dstv, rows, sem):
        c = lax.axis_index("c")
        s = lax.axis_index("s")
        pltpu.sync_copy(zeros.at[pl.ds(s * RPT, RPT)],
                        acc.at[pl.ds(s * RPT, RPT)])
        plsc.subcore_barrier()

        base = s * NCHUNK

        def body(i, carry):
            pltpu.sync_copy(srcall.at[c, base + i], srcv)
            pltpu.sync_copy(dst2d.at[base + i], dstv)
            pltpu.async_copy(xcat.at[srcv], rows, sem).wait()
            pltpu.sync_copy(rows, acc.at[dstv], add=True)
            return carry

        lax.fori_loop(0, NCHUNK, body, 0)
        plsc.subcore_barrier()
        pltpu.sync_copy(acc.at[pl.ds(s * RPT, RPT)],
                        out.at[pl.ds(c * N + s * RPT, RPT)])

    return agg


_agg = _make_agg()


BN = 400


def _tc1_body(p0, p1, xr, wl, wr, br, o):
    cnt = jnp.maximum(p1[:, HALF:HALF + 1], 1.0)
    ml = p0[:, :HALF] / cnt
    mh = p1[:, :HALF] / cnt
    z = (jnp.dot(ml, wl[:HALF, :], preferred_element_type=jnp.float32)
         + jnp.dot(mh, wl[HALF:, :], preferred_element_type=jnp.float32)
         + jnp.dot(xr[...], wr[...], preferred_element_type=jnp.float32)
         + br[...])
    h = jnp.maximum(z, 0.0)
    pad = jnp.zeros((BN, W - HALF), jnp.float32)
    o[0] = jnp.concatenate([h[:, :HALF], pad], axis=1)
    o[1] = jnp.concatenate([h[:, HALF:], pad], axis=1)


def _tc1(agg1, x, Wl1, Wr1, b1):
    nb = N // BN
    return pl.pallas_call(
        _tc1_body,
        grid=(nb,),
        in_specs=[
            pl.BlockSpec((BN, W), lambda i: (i, 0)),
            pl.BlockSpec((BN, W), lambda i: (nb + i, 0)),
            pl.BlockSpec((BN, D), lambda i: (i, 0)),
            pl.BlockSpec((D, D), lambda i: (0, 0)),
            pl.BlockSpec((D, D), lambda i: (0, 0)),
            pl.BlockSpec((1, D), lambda i: (0, 0)),
        ],
        out_specs=pl.BlockSpec((2, BN, W), lambda i: (0, i, 0)),
        out_shape=jax.ShapeDtypeStruct((2, N, W), jnp.float32),
    )(agg1, agg1, x, Wl1, Wr1, b1)


def _tc2_body(a0, a1, c1, h2, wl, wr, br, o):
    cnt = jnp.maximum(c1[:, HALF:HALF + 1], 1.0)
    ml = a0[:, :HALF] / cnt
    mh = a1[:, :HALF] / cnt
    h0 = h2[0, :, :HALF]
    h1 = h2[1, :, :HALF]
    z = (jnp.dot(ml, wl[:HALF, :], preferred_element_type=jnp.float32)
         + jnp.dot(mh, wl[HALF:, :], preferred_element_type=jnp.float32)
         + jnp.dot(h0, wr[:HALF, :], preferred_element_type=jnp.float32)
         + jnp.dot(h1, wr[HALF:, :], preferred_element_type=jnp.float32)
         + br[...])
    o[...] = z


def _tc2(agg2, agg1, h2, Wl2, Wr2, b2):
    nb = N // BN
    return pl.pallas_call(
        _tc2_body,
        grid=(nb,),
        in_specs=[
            pl.BlockSpec((BN, W), lambda i: (i, 0)),
            pl.BlockSpec((BN, W), lambda i: (nb + i, 0)),
            pl.BlockSpec((BN, W), lambda i: (nb + i, 0)),
            pl.BlockSpec((2, BN, W), lambda i: (0, i, 0)),
            pl.BlockSpec((D, D), lambda i: (0, 0)),
            pl.BlockSpec((D, D), lambda i: (0, 0)),
            pl.BlockSpec((1, D), lambda i: (0, 0)),
        ],
        out_specs=pl.BlockSpec((BN, D), lambda i: (i, 0)),
        out_shape=jax.ShapeDtypeStruct((N, D), jnp.float32),
    )(agg2, agg2, agg1, h2, Wl2, Wr2, b2)


def kernel(x, edge_index, Wl1, Wr1, b1, Wl2, Wr2, b2):
    src = edge_index[0].astype(jnp.int32).reshape(NCHUNK_TOT, K)
    dst2d = edge_index[1].astype(jnp.int32).reshape(NCHUNK_TOT, K)
    srcall = jnp.stack([src, src + N])
    part0 = jnp.pad(x[:, :HALF], ((0, 0), (0, W - HALF)))
    part1 = jnp.concatenate(
        [x[:, HALF:], jnp.ones((N, 1), jnp.float32),
         jnp.zeros((N, W - HALF - 1), jnp.float32)], axis=1)
    xcat = jnp.concatenate([part0, part1], axis=0)
    zeros = jnp.zeros((N, W), jnp.float32)

    agg1 = _agg(xcat, srcall, dst2d, zeros)
    h2 = _tc1(agg1, x, Wl1, Wr1, b1.reshape(1, D))
    hcat = h2.reshape(2 * N, W)
    agg2 = _agg(hcat, srcall, dst2d, zeros)
    return _tc2(agg2, agg1, h2, Wl2, Wr2, b2.reshape(1, D))


# speedup vs baseline: 3.4330x; 1.0680x over previous
import functools

import jax
import jax.numpy as jnp
from jax import lax
from jax.experimental import pallas as pl
from jax.experimental.pallas import tpu as pltpu
from jax.experimental.pallas import tpu_sc as plsc

N = 10000
E = 320000
D = 128
HALF = 64
W = 80
K = 128
NS = 16
NC = 2
G = 16
NSUP = 10
NCHUNK = NSUP * G
EP = NCHUNK * NS * K
NCHUNK_TOT = EP // K
NB = 4
RPT = N // NS
ACCR = N + NS
RPTZ = ACCR // NS


def _make_agg():
    mesh = plsc.VectorSubcoreMesh(core_axis_name="c", subcore_axis_name="s")

    @functools.partial(
        pl.kernel,
        mesh=mesh,
        compiler_params=pltpu.CompilerParams(use_tc_tiling_on_sc=False),
        out_type=jax.ShapeDtypeStruct((2 * N, W), jnp.float32),
        scratch_types=[
            pltpu.VMEM_SHARED((ACCR, W), jnp.float32),
            pltpu.VMEM((G, K), jnp.int32),
            pltpu.VMEM((G, K), jnp.int32),
            [pltpu.VMEM((K, W), jnp.float32) for _ in range(NB)],
            pltpu.SemaphoreType.DMA((NB,)),
            pltpu.SemaphoreType.DMA((NB,)),
        ],
    )
    def agg(xcat, srcall, dst2d, zeros, out, acc, srcb, dstb, rows, gsem, ssem):
        c = lax.axis_index("c")
        s = lax.axis_index("s")
        pltpu.sync_copy(zeros.at[pl.ds(s * RPTZ, RPTZ)],
                        acc.at[pl.ds(s * RPTZ, RPTZ)])
        plsc.subcore_barrier()

        base = s * NCHUNK

        def super_body(g, carry):
            pltpu.sync_copy(srcall.at[c, pl.ds(base + g * G, G)], srcb)
            pltpu.sync_copy(dst2d.at[pl.ds(base + g * G, G)], dstb)

            def gather(j):
                b = j % NB
                return pltpu.async_copy(
                    xcat.at[srcb.at[j]], rows[b], gsem.at[b])

            gd = [None] * G
            sd = [None] * G
            for j in range(NB - 1):
                gd[j] = gather(j)
            for j in range(G):
                b = j % NB
                gd[j].wait()
                sd[j] = pltpu.async_copy(
                    rows[b], acc.at[dstb.at[j]], ssem.at[b], add=True)
                nxt = j + NB - 1
                if nxt < G:
                    if nxt - NB >= 0:
                        sd[nxt - NB].wait()
                    gd[nxt] = gather(nxt)
            for j in range(G - NB, G):
                sd[j].wait()
            return carry

        lax.fori_loop(0, NSUP, super_body, 0)
        plsc.subcore_barrier()
        pltpu.sync_copy(acc.at[pl.ds(s * RPT, RPT)],
                        out.at[pl.ds(c * N + s * RPT, RPT)])

    return agg


_agg = _make_agg()


BN = 400


def _tc1_body(p0, p1, xr, wl, wr, br, o):
    cnt = jnp.maximum(p1[:, HALF:HALF + 1], 1.0)
    ml = p0[:, :HALF] / cnt
    mh = p1[:, :HALF] / cnt
    z = (jnp.dot(ml, wl[:HALF, :], preferred_element_type=jnp.float32)
         + jnp.dot(mh, wl[HALF:, :], preferred_element_type=jnp.float32)
         + jnp.dot(xr[...], wr[...], preferred_element_type=jnp.float32)
         + br[...])
    h = jnp.maximum(z, 0.0)
    pad = jnp.zeros((BN, W - HALF), jnp.float32)
    o[0] = jnp.concatenate([h[:, :HALF], pad], axis=1)
    o[1] = jnp.concatenate([h[:, HALF:], pad], axis=1)


def _tc1(agg1, x, Wl1, Wr1, b1):
    nb = N // BN
    return pl.pallas_call(
        _tc1_body,
        grid=(nb,),
        in_specs=[
            pl.BlockSpec((BN, W), lambda i: (i, 0)),
            pl.BlockSpec((BN, W), lambda i: (nb + i, 0)),
            pl.BlockSpec((BN, D), lambda i: (i, 0)),
            pl.BlockSpec((D, D), lambda i: (0, 0)),
            pl.BlockSpec((D, D), lambda i: (0, 0)),
            pl.BlockSpec((1, D), lambda i: (0, 0)),
        ],
        out_specs=pl.BlockSpec((2, BN, W), lambda i: (0, i, 0)),
        out_shape=jax.ShapeDtypeStruct((2, N, W), jnp.float32),
    )(agg1, agg1, x, Wl1, Wr1, b1)


def _tc2_body(a0, a1, c1, h2, wl, wr, br, o):
    cnt = jnp.maximum(c1[:, HALF:HALF + 1], 1.0)
    ml = a0[:, :HALF] / cnt
    mh = a1[:, :HALF] / cnt
    h0 = h2[0, :, :HALF]
    h1 = h2[1, :, :HALF]
    z = (jnp.dot(ml, wl[:HALF, :], preferred_element_type=jnp.float32)
         + jnp.dot(mh, wl[HALF:, :], preferred_element_type=jnp.float32)
         + jnp.dot(h0, wr[:HALF, :], preferred_element_type=jnp.float32)
         + jnp.dot(h1, wr[HALF:, :], preferred_element_type=jnp.float32)
         + br[...])
    o[...] = z


def _tc2(agg2, agg1, h2, Wl2, Wr2, b2):
    nb = N // BN
    return pl.pallas_call(
        _tc2_body,
        grid=(nb,),
        in_specs=[
            pl.BlockSpec((BN, W), lambda i: (i, 0)),
            pl.BlockSpec((BN, W), lambda i: (nb + i, 0)),
            pl.BlockSpec((BN, W), lambda i: (nb + i, 0)),
            pl.BlockSpec((2, BN, W), lambda i: (0, i, 0)),
            pl.BlockSpec((D, D), lambda i: (0, 0)),
            pl.BlockSpec((D, D), lambda i: (0, 0)),
            pl.BlockSpec((1, D), lambda i: (0, 0)),
        ],
        out_specs=pl.BlockSpec((BN, D), lambda i: (i, 0)),
        out_shape=jax.ShapeDtypeStruct((N, D), jnp.float32),
    )(agg2, agg2, agg1, h2, Wl2, Wr2, b2)


def kernel(x, edge_index, Wl1, Wr1, b1, Wl2, Wr2, b2):
    src = edge_index[0].astype(jnp.int32)
    dst = edge_index[1].astype(jnp.int32)
    src_p = jnp.concatenate(
        [src, jnp.full((EP - E,), 2 * N, jnp.int32)]).reshape(NCHUNK_TOT, K)
    dst2d = jnp.concatenate(
        [dst, jnp.full((EP - E,), N, jnp.int32)]).reshape(NCHUNK_TOT, K)
    srcall = jnp.stack([src_p, jnp.where(src_p < 2 * N, src_p + N, src_p)])
    part0 = jnp.pad(x[:, :HALF], ((0, 0), (0, W - HALF)))
    part1 = jnp.concatenate(
        [x[:, HALF:], jnp.ones((N, 1), jnp.float32),
         jnp.zeros((N, W - HALF - 1), jnp.float32)], axis=1)
    xcat = jnp.concatenate(
        [part0, part1, jnp.zeros((8, W), jnp.float32)], axis=0)
    zeros = jnp.zeros((ACCR, W), jnp.float32)

    agg1 = _agg(xcat, srcall, dst2d, zeros)
    h2 = _tc1(agg1, x, Wl1, Wr1, b1.reshape(1, D))
    hcat = h2.reshape(2 * N, W)
    agg2 = _agg(hcat, srcall, dst2d, zeros)
    return _tc2(agg2, agg1, h2, Wl2, Wr2, b2.reshape(1, D))


# speedup vs baseline: 7.3660x; 2.1456x over previous
import functools

import jax
import jax.numpy as jnp
from jax import lax
from jax.experimental import pallas as pl
from jax.experimental.pallas import tpu as pltpu
from jax.experimental.pallas import tpu_sc as plsc

N = 10000
E = 320000
D = 128
HALF = 64
W = 80
K = 128
NS = 16
NC = 2
G = 16
NSUP = 10
NCHUNK = NSUP * G
EP = NCHUNK * NS * K
NCHUNK_TOT = EP // K
NB = 2
RPT = N // NS
ACCR = N + NS
RPTZ = ACCR // NS


def _make_agg():
    mesh = plsc.VectorSubcoreMesh(core_axis_name="c", subcore_axis_name="s")

    @functools.partial(
        pl.kernel,
        mesh=mesh,
        compiler_params=pltpu.CompilerParams(use_tc_tiling_on_sc=False),
        out_type=jax.ShapeDtypeStruct((2 * N, W), jnp.float32),
        scratch_types=[
            pltpu.VMEM_SHARED((ACCR, W), jnp.float32),
            pltpu.VMEM_SHARED((ACCR, W), jnp.float32),
            pltpu.VMEM((G, K), jnp.int32),
            pltpu.VMEM((G, K), jnp.int32),
            [pltpu.VMEM((K, W), jnp.float32) for _ in range(NB)],
            pltpu.SemaphoreType.DMA((NB,)),
            pltpu.SemaphoreType.DMA((NB,)),
        ],
    )
    def agg(xstk, srcall, dst2d, zeros, out, acc, xs, srcb, dstb, rows, gsem, ssem):
        c = lax.axis_index("c")
        s = lax.axis_index("s")
        pltpu.sync_copy(zeros.at[pl.ds(s * RPTZ, RPTZ)],
                        acc.at[pl.ds(s * RPTZ, RPTZ)])
        pltpu.sync_copy(xstk.at[c, pl.ds(s * RPTZ, RPTZ)],
                        xs.at[pl.ds(s * RPTZ, RPTZ)])
        plsc.subcore_barrier()

        base = s * NCHUNK

        def super_body(g, carry):
            pltpu.sync_copy(srcall.at[c, pl.ds(base + g * G, G)], srcb)
            pltpu.sync_copy(dst2d.at[pl.ds(base + g * G, G)], dstb)

            def gather(j):
                b = j % NB
                return pltpu.async_copy(
                    xs.at[srcb.at[j]], rows[b], gsem.at[b])

            gd = [None] * G
            sd = [None] * G
            for j in range(NB - 1):
                gd[j] = gather(j)
            for j in range(G):
                b = j % NB
                gd[j].wait()
                sd[j] = pltpu.async_copy(
                    rows[b], acc.at[dstb.at[j]], ssem.at[b], add=True)
                nxt = j + NB - 1
                if nxt < G:
                    if nxt - NB >= 0:
                        sd[nxt - NB].wait()
                    gd[nxt] = gather(nxt)
            for j in range(G - NB, G):
                sd[j].wait()
            return carry

        lax.fori_loop(0, NSUP, super_body, 0)
        plsc.subcore_barrier()
        pltpu.sync_copy(acc.at[pl.ds(s * RPT, RPT)],
                        out.at[pl.ds(c * N + s * RPT, RPT)])

    return agg


_agg = _make_agg()


BN = 400


def _tc1_body(p0, p1, xr, wl, wr, br, o):
    cnt = jnp.maximum(p1[:, HALF:HALF + 1], 1.0)
    ml = p0[:, :HALF] / cnt
    mh = p1[:, :HALF] / cnt
    z = (jnp.dot(ml, wl[:HALF, :], preferred_element_type=jnp.float32)
         + jnp.dot(mh, wl[HALF:, :], preferred_element_type=jnp.float32)
         + jnp.dot(xr[...], wr[...], preferred_element_type=jnp.float32)
         + br[...])
    h = jnp.maximum(z, 0.0)
    pad = jnp.zeros((BN, W - HALF), jnp.float32)
    o[0] = jnp.concatenate([h[:, :HALF], pad], axis=1)
    o[1] = jnp.concatenate([h[:, HALF:], pad], axis=1)


def _tc1(agg1, x, Wl1, Wr1, b1):
    nb = N // BN
    return pl.pallas_call(
        _tc1_body,
        grid=(nb,),
        in_specs=[
            pl.BlockSpec((BN, W), lambda i: (i, 0)),
            pl.BlockSpec((BN, W), lambda i: (nb + i, 0)),
            pl.BlockSpec((BN, D), lambda i: (i, 0)),
            pl.BlockSpec((D, D), lambda i: (0, 0)),
            pl.BlockSpec((D, D), lambda i: (0, 0)),
            pl.BlockSpec((1, D), lambda i: (0, 0)),
        ],
        out_specs=pl.BlockSpec((2, BN, W), lambda i: (0, i, 0)),
        out_shape=jax.ShapeDtypeStruct((2, N, W), jnp.float32),
    )(agg1, agg1, x, Wl1, Wr1, b1)


def _tc2_body(a0, a1, c1, h2, wl, wr, br, o):
    cnt = jnp.maximum(c1[:, HALF:HALF + 1], 1.0)
    ml = a0[:, :HALF] / cnt
    mh = a1[:, :HALF] / cnt
    h0 = h2[0, :, :HALF]
    h1 = h2[1, :, :HALF]
    z = (jnp.dot(ml, wl[:HALF, :], preferred_element_type=jnp.float32)
         + jnp.dot(mh, wl[HALF:, :], preferred_element_type=jnp.float32)
         + jnp.dot(h0, wr[:HALF, :], preferred_element_type=jnp.float32)
         + jnp.dot(h1, wr[HALF:, :], preferred_element_type=jnp.float32)
         + br[...])
    o[...] = z


def _tc2(agg2, agg1, h2, Wl2, Wr2, b2):
    nb = N // BN
    return pl.pallas_call(
        _tc2_body,
        grid=(nb,),
        in_specs=[
            pl.BlockSpec((BN, W), lambda i: (i, 0)),
            pl.BlockSpec((BN, W), lambda i: (nb + i, 0)),
            pl.BlockSpec((BN, W), lambda i: (nb + i, 0)),
            pl.BlockSpec((2, BN, W), lambda i: (0, i, 0)),
            pl.BlockSpec((D, D), lambda i: (0, 0)),
            pl.BlockSpec((D, D), lambda i: (0, 0)),
            pl.BlockSpec((1, D), lambda i: (0, 0)),
        ],
        out_specs=pl.BlockSpec((BN, D), lambda i: (i, 0)),
        out_shape=jax.ShapeDtypeStruct((N, D), jnp.float32),
    )(agg2, agg2, agg1, h2, Wl2, Wr2, b2)


def kernel(x, edge_index, Wl1, Wr1, b1, Wl2, Wr2, b2):
    src = edge_index[0].astype(jnp.int32)
    dst = edge_index[1].astype(jnp.int32)
    src_p = jnp.concatenate(
        [src, jnp.full((EP - E,), N, jnp.int32)]).reshape(NCHUNK_TOT, K)
    dst2d = jnp.concatenate(
        [dst, jnp.full((EP - E,), N, jnp.int32)]).reshape(NCHUNK_TOT, K)
    srcall = jnp.stack([src_p, src_p])
    part0 = jnp.pad(x[:, :HALF], ((0, 0), (0, W - HALF)))
    part1 = jnp.concatenate(
        [x[:, HALF:], jnp.ones((N, 1), jnp.float32),
         jnp.zeros((N, W - HALF - 1), jnp.float32)], axis=1)
    zpad = jnp.zeros((ACCR - N, W), jnp.float32)
    xstk = jnp.stack([jnp.concatenate([part0, zpad]),
                      jnp.concatenate([part1, zpad])])
    zeros = jnp.zeros((ACCR, W), jnp.float32)

    agg1 = _agg(xstk, srcall, dst2d, zeros)
    h2 = _tc1(agg1, x, Wl1, Wr1, b1.reshape(1, D))
    hstk = h2.reshape(2, N, W)
    hstk = jnp.concatenate([hstk, jnp.zeros((2, ACCR - N, W), jnp.float32)], axis=1)
    agg2 = _agg(hstk, srcall, dst2d, zeros)
    return _tc2(agg2, agg1, h2, Wl2, Wr2, b2.reshape(1, D))


# speedup vs baseline: 8.4766x; 1.1508x over previous
import functools

import jax
import jax.numpy as jnp
from jax import lax
from jax.experimental import pallas as pl
from jax.experimental.pallas import tpu as pltpu
from jax.experimental.pallas import tpu_sc as plsc

N = 10000
E = 320000
D = 128
HALF = 64
CW = 16
K = 128
NS = 16
G = 16
NSUP = 10
NCHUNK = NSUP * G
EP = NCHUNK * NS * K
NCHUNK_TOT = EP // K
NB = 4
RPT = N // NS
ACCR = N + NS
RPTZ = ACCR // NS


def _make_agg(with_cnt):
    mesh = plsc.VectorSubcoreMesh(core_axis_name="c", subcore_axis_name="s")

    out_type = [jax.ShapeDtypeStruct((2, N, HALF), jnp.float32)]
    scratch = [
        pltpu.VMEM_SHARED((ACCR, HALF), jnp.float32),
        pltpu.VMEM_SHARED((ACCR, HALF), jnp.float32),
        pltpu.VMEM((G, 2, K), jnp.int32),
        [pltpu.VMEM((K, HALF), jnp.float32) for _ in range(NB)],
        pltpu.SemaphoreType.DMA((NB,)),
        pltpu.SemaphoreType.DMA((NB,)),
    ]
    if with_cnt:
        out_type.append(jax.ShapeDtypeStruct((N, CW), jnp.float32))
        scratch += [
            pltpu.VMEM_SHARED((ACCR, CW), jnp.float32),
            pltpu.VMEM((K, CW), jnp.float32),
            pltpu.SemaphoreType.DMA((NB,)),
        ]

    @functools.partial(
        pl.kernel,
        mesh=mesh,
        compiler_params=pltpu.CompilerParams(use_tc_tiling_on_sc=False),
        out_type=out_type,
        scratch_types=scratch,
    )
    def agg(*args):
        if with_cnt:
            (xstk, edges, zeros, zeros16, ones_hbm, out, outc, acc, xs, idx,
             rows, gsem, ssem, cntacc, ones_v, csem) = args
        else:
            xstk, edges, zeros, out, acc, xs, idx, rows, gsem, ssem = args
        c = lax.axis_index("c")
        s = lax.axis_index("s")
        pltpu.sync_copy(zeros.at[pl.ds(s * RPTZ, RPTZ)],
                        acc.at[pl.ds(s * RPTZ, RPTZ)])
        pltpu.sync_copy(xstk.at[c, pl.ds(s * RPTZ, RPTZ)],
                        xs.at[pl.ds(s * RPTZ, RPTZ)])
        if with_cnt:
            pltpu.sync_copy(zeros16.at[pl.ds(s * RPTZ, RPTZ)],
                            cntacc.at[pl.ds(s * RPTZ, RPTZ)])
            pltpu.sync_copy(ones_hbm, ones_v)
        plsc.subcore_barrier()

        base = s * NSUP

        def super_body(g, carry):
            pltpu.sync_copy(edges.at[pl.ds((base + g) * G, G)], idx)

            def gather(j):
                b = j % NB
                return pltpu.async_copy(
                    xs.at[idx.at[j, 0]], rows[b], gsem.at[b])

            gd = [None] * G
            sd = [None] * G
            cd = [None] * G
            for j in range(NB - 1):
                gd[j] = gather(j)
            for j in range(G):
                b = j % NB
                gd[j].wait()
                sd[j] = pltpu.async_copy(
                    rows[b], acc.at[idx.at[j, 1]], ssem.at[b], add=True)
                if with_cnt:
                    cd[j] = pltpu.async_copy(
                        ones_v, cntacc.at[idx.at[j, 1]], csem.at[b], add=True)
                nxt = j + NB - 1
                if nxt < G:
                    if nxt - NB >= 0:
                        sd[nxt - NB].wait()
                        if with_cnt:
                            cd[nxt - NB].wait()
                    gd[nxt] = gather(nxt)
            for j in range(G - NB, G):
                sd[j].wait()
                if with_cnt:
                    cd[j].wait()
            return carry

        lax.fori_loop(0, NSUP, super_body, 0)
        plsc.subcore_barrier()
        pltpu.sync_copy(acc.at[pl.ds(s * RPT, RPT)],
                        out.at[c, pl.ds(s * RPT, RPT)])
        if with_cnt:
            @pl.when(c == 0)
            def _():
                pltpu.sync_copy(cntacc.at[pl.ds(s * RPT, RPT)],
                                outc.at[pl.ds(s * RPT, RPT)])

    return agg


_agg_cnt = _make_agg(True)
_agg = _make_agg(False)


BN = 400


def _tc1_body(p, ct, xr, wl, wr, br, o):
    cnt = jnp.maximum(ct[:, :1], 1.0)
    ml = p[0] / cnt
    mh = p[1] / cnt
    z = (jnp.dot(ml, wl[:HALF, :], preferred_element_type=jnp.float32)
         + jnp.dot(mh, wl[HALF:, :], preferred_element_type=jnp.float32)
         + jnp.dot(xr[...], wr[...], preferred_element_type=jnp.float32)
         + br[...])
    h = jnp.maximum(z, 0.0)
    o[0] = h[:, :HALF]
    o[1] = h[:, HALF:]


def _tc1(sums, cnt, x, Wl1, Wr1, b1):
    return pl.pallas_call(
        _tc1_body,
        grid=(N // BN,),
        in_specs=[
            pl.BlockSpec((2, BN, HALF), lambda i: (0, i, 0)),
            pl.BlockSpec((BN, CW), lambda i: (i, 0)),
            pl.BlockSpec((BN, D), lambda i: (i, 0)),
            pl.BlockSpec((D, D), lambda i: (0, 0)),
            pl.BlockSpec((D, D), lambda i: (0, 0)),
            pl.BlockSpec((1, D), lambda i: (0, 0)),
        ],
        out_specs=pl.BlockSpec((2, BN, HALF), lambda i: (0, i, 0)),
        out_shape=jax.ShapeDtypeStruct((2, ACCR, HALF), jnp.float32),
    )(sums, cnt, x, Wl1, Wr1, b1)


def _tc2_body(a, ct, h2, wl, wr, br, o):
    cnt = jnp.maximum(ct[:, :1], 1.0)
    ml = a[0] / cnt
    mh = a[1] / cnt
    z = (jnp.dot(ml, wl[:HALF, :], preferred_element_type=jnp.float32)
         + jnp.dot(mh, wl[HALF:, :], preferred_element_type=jnp.float32)
         + jnp.dot(h2[0], wr[:HALF, :], preferred_element_type=jnp.float32)
         + jnp.dot(h2[1], wr[HALF:, :], preferred_element_type=jnp.float32)
         + br[...])
    o[...] = z


def _tc2(agg2, cnt, h2, Wl2, Wr2, b2):
    return pl.pallas_call(
        _tc2_body,
        grid=(N // BN,),
        in_specs=[
            pl.BlockSpec((2, BN, HALF), lambda i: (0, i, 0)),
            pl.BlockSpec((BN, CW), lambda i: (i, 0)),
            pl.BlockSpec((2, BN, HALF), lambda i: (0, i, 0)),
            pl.BlockSpec((D, D), lambda i: (0, 0)),
            pl.BlockSpec((D, D), lambda i: (0, 0)),
            pl.BlockSpec((1, D), lambda i: (0, 0)),
        ],
        out_specs=pl.BlockSpec((BN, D), lambda i: (i, 0)),
        out_shape=jax.ShapeDtypeStruct((N, D), jnp.float32),
    )(agg2, cnt, h2, Wl2, Wr2, b2)


def kernel(x, edge_index, Wl1, Wr1, b1, Wl2, Wr2, b2):
    src = edge_index[0].astype(jnp.int32)
    dst = edge_index[1].astype(jnp.int32)
    src_p = jnp.concatenate(
        [src, jnp.zeros((EP - E,), jnp.int32)]).reshape(NCHUNK_TOT, K)
    dst_p = jnp.concatenate(
        [dst, jnp.full((EP - E,), N, jnp.int32)]).reshape(NCHUNK_TOT, K)
    edges = jnp.stack([src_p, dst_p], axis=1)
    rpad = ((0, ACCR - N), (0, 0))
    xstk = jnp.stack([jnp.pad(x[:, :HALF], rpad),
                      jnp.pad(x[:, HALF:], rpad)])
    zeros = jnp.zeros((ACCR, HALF), jnp.float32)
    zeros16 = jnp.zeros((ACCR, CW), jnp.float32)
    ones16 = jnp.ones((K, CW), jnp.float32)

    sums, cnt = _agg_cnt(xstk, edges, zeros, zeros16, ones16)
    h2 = _tc1(sums, cnt, x, Wl1, Wr1, b1.reshape(1, D))
    (agg2,) = _agg(h2, edges, zeros)
    return _tc2(agg2, cnt, h2, Wl2, Wr2, b2.reshape(1, D))


# speedup vs baseline: 8.9524x; 1.0561x over previous
import functools

import jax
import jax.numpy as jnp
from jax import lax
from jax.experimental import pallas as pl
from jax.experimental.pallas import tpu as pltpu
from jax.experimental.pallas import tpu_sc as plsc

N = 10000
E = 320000
D = 128
HALF = 64
CW = 16
K = 128
NS = 16
G = 16
NSUP = 10
NCHUNK = NSUP * G
EP = NCHUNK * NS * K
NCHUNK_TOT = EP // K
NB = 4
RPT = N // NS
ACCR = N + NS
RPTZ = ACCR // NS


def _make_agg(with_cnt, nb):
    mesh = plsc.VectorSubcoreMesh(core_axis_name="c", subcore_axis_name="s")

    out_type = [jax.ShapeDtypeStruct((2, N, HALF), jnp.float32)]
    scratch = [
        pltpu.VMEM_SHARED((ACCR, HALF), jnp.float32),
        pltpu.VMEM_SHARED((ACCR, HALF), jnp.float32),
        pltpu.VMEM((G, 2, K), jnp.int32),
        [pltpu.VMEM((K, HALF), jnp.float32) for _ in range(nb)],
        pltpu.SemaphoreType.DMA((nb,)),
        pltpu.SemaphoreType.DMA((nb,)),
    ]
    if with_cnt:
        out_type.append(jax.ShapeDtypeStruct((2, N, CW), jnp.float32))
        scratch += [
            pltpu.VMEM_SHARED((ACCR, CW), jnp.float32),
            pltpu.VMEM((G // 2, K), jnp.int32),
            pltpu.VMEM((K, CW), jnp.float32),
            pltpu.SemaphoreType.DMA((nb,)),
        ]

    @functools.partial(
        pl.kernel,
        mesh=mesh,
        compiler_params=pltpu.CompilerParams(use_tc_tiling_on_sc=False),
        out_type=out_type,
        scratch_types=scratch,
    )
    def agg(*args):
        if with_cnt:
            (xstk, edges, cdst, zeros, zeros16, ones_hbm, out, outc, acc, xs,
             idx, rows, gsem, ssem, cntacc, cidx, ones_v, csem) = args
        else:
            xstk, edges, zeros, out, acc, xs, idx, rows, gsem, ssem = args
        c = lax.axis_index("c")
        s = lax.axis_index("s")
        pltpu.sync_copy(zeros.at[pl.ds(s * RPTZ, RPTZ)],
                        acc.at[pl.ds(s * RPTZ, RPTZ)])
        pltpu.sync_copy(xstk.at[c, pl.ds(s * RPTZ, RPTZ)],
                        xs.at[pl.ds(s * RPTZ, RPTZ)])
        if with_cnt:
            pltpu.sync_copy(zeros16.at[pl.ds(s * RPTZ, RPTZ)],
                            cntacc.at[pl.ds(s * RPTZ, RPTZ)])
            pltpu.sync_copy(ones_hbm, ones_v)
        plsc.subcore_barrier()

        base = s * NSUP

        def super_body(g, carry):
            pltpu.sync_copy(edges.at[pl.ds((base + g) * G, G)], idx)
            if with_cnt:
                pltpu.sync_copy(
                    cdst.at[c, pl.ds(s * (NSUP * G // 2) + g * (G // 2),
                                     G // 2)], cidx)

            def gather(j):
                b = j % nb
                return pltpu.async_copy(
                    xs.at[idx.at[j, 0]], rows[b], gsem.at[b])

            def wait_cnt(j):
                if with_cnt and j % 2 == 0:
                    cd[j].wait()

            gd = [None] * G
            sd = [None] * G
            cd = [None] * G
            for j in range(nb - 1):
                gd[j] = gather(j)
            for j in range(G):
                b = j % nb
                gd[j].wait()
                sd[j] = pltpu.async_copy(
                    rows[b], acc.at[idx.at[j, 1]], ssem.at[b], add=True)
                if with_cnt and j % 2 == 0:
                    cd[j] = pltpu.async_copy(
                        ones_v, cntacc.at[cidx.at[j // 2]],
                        csem.at[(j // 2) % nb], add=True)
                nxt = j + nb - 1
                if nxt < G:
                    if nxt - nb >= 0:
                        sd[nxt - nb].wait()
                        wait_cnt(nxt - nb)
                    gd[nxt] = gather(nxt)
            for j in range(G - nb, G):
                sd[j].wait()
                wait_cnt(j)
            return carry

        lax.fori_loop(0, NSUP, super_body, 0)
        plsc.subcore_barrier()
        pltpu.sync_copy(acc.at[pl.ds(s * RPT, RPT)],
                        out.at[c, pl.ds(s * RPT, RPT)])
        if with_cnt:
            pltpu.sync_copy(cntacc.at[pl.ds(s * RPT, RPT)],
                            outc.at[c, pl.ds(s * RPT, RPT)])

    return agg


_agg_cnt = _make_agg(True, 4)
_agg = _make_agg(False, 5)


BN = 2000


def _tc1_body(p, ct, xr, wl, wr, br, o):
    cnt = jnp.maximum(ct[0, :, :1] + ct[1, :, :1], 1.0)
    ml = p[0] / cnt
    mh = p[1] / cnt
    z = (jnp.dot(ml, wl[:HALF, :], preferred_element_type=jnp.float32)
         + jnp.dot(mh, wl[HALF:, :], preferred_element_type=jnp.float32)
         + jnp.dot(xr[...], wr[...], preferred_element_type=jnp.float32)
         + br[...])
    h = jnp.maximum(z, 0.0)
    o[0] = h[:, :HALF]
    o[1] = h[:, HALF:]


def _tc1(sums, cnt, x, Wl1, Wr1, b1):
    return pl.pallas_call(
        _tc1_body,
        grid=(N // BN,),
        in_specs=[
            pl.BlockSpec((2, BN, HALF), lambda i: (0, i, 0)),
            pl.BlockSpec((2, BN, CW), lambda i: (0, i, 0)),
            pl.BlockSpec((BN, D), lambda i: (i, 0)),
            pl.BlockSpec((D, D), lambda i: (0, 0)),
            pl.BlockSpec((D, D), lambda i: (0, 0)),
            pl.BlockSpec((1, D), lambda i: (0, 0)),
        ],
        out_specs=pl.BlockSpec((2, BN, HALF), lambda i: (0, i, 0)),
        out_shape=jax.ShapeDtypeStruct((2, ACCR, HALF), jnp.float32),
    )(sums, cnt, x, Wl1, Wr1, b1)


def _tc2_body(a, ct, h2, wl, wr, br, o):
    cnt = jnp.maximum(ct[0, :, :1] + ct[1, :, :1], 1.0)
    ml = a[0] / cnt
    mh = a[1] / cnt
    z = (jnp.dot(ml, wl[:HALF, :], preferred_element_type=jnp.float32)
         + jnp.dot(mh, wl[HALF:, :], preferred_element_type=jnp.float32)
         + jnp.dot(h2[0], wr[:HALF, :], preferred_element_type=jnp.float32)
         + jnp.dot(h2[1], wr[HALF:, :], preferred_element_type=jnp.float32)
         + br[...])
    o[...] = z


def _tc2(agg2, cnt, h2, Wl2, Wr2, b2):
    return pl.pallas_call(
        _tc2_body,
        grid=(N // BN,),
        in_specs=[
            pl.BlockSpec((2, BN, HALF), lambda i: (0, i, 0)),
            pl.BlockSpec((2, BN, CW), lambda i: (0, i, 0)),
            pl.BlockSpec((2, BN, HALF), lambda i: (0, i, 0)),
            pl.BlockSpec((D, D), lambda i: (0, 0)),
            pl.BlockSpec((D, D), lambda i: (0, 0)),
            pl.BlockSpec((1, D), lambda i: (0, 0)),
        ],
        out_specs=pl.BlockSpec((BN, D), lambda i: (i, 0)),
        out_shape=jax.ShapeDtypeStruct((N, D), jnp.float32),
    )(agg2, cnt, h2, Wl2, Wr2, b2)


def kernel(x, edge_index, Wl1, Wr1, b1, Wl2, Wr2, b2):
    src = edge_index[0].astype(jnp.int32)
    dst = edge_index[1].astype(jnp.int32)
    src_p = jnp.concatenate(
        [src, jnp.zeros((EP - E,), jnp.int32)]).reshape(NCHUNK_TOT, K)
    dst_p = jnp.concatenate(
        [dst, jnp.full((EP - E,), N, jnp.int32)]).reshape(NCHUNK_TOT, K)
    edges = jnp.stack([src_p, dst_p], axis=1)
    cdst = dst_p.reshape(2, NCHUNK_TOT // 2, K)
    rpad = ((0, ACCR - N), (0, 0))
    xstk = jnp.stack([jnp.pad(x[:, :HALF], rpad),
                      jnp.pad(x[:, HALF:], rpad)])
    zeros = jnp.zeros((ACCR, HALF), jnp.float32)
    zeros16 = jnp.zeros((ACCR, CW), jnp.float32)
    ones16 = jnp.ones((K, CW), jnp.float32)

    sums, cnt = _agg_cnt(xstk, edges, cdst, zeros, zeros16, ones16)
    h2 = _tc1(sums, cnt, x, Wl1, Wr1, b1.reshape(1, D))
    (agg2,) = _agg(h2, edges, zeros)
    return _tc2(agg2, cnt, h2, Wl2, Wr2, b2.reshape(1, D))


# speedup vs baseline: 10.1202x; 1.1304x over previous
import functools

import jax
import jax.numpy as jnp
from jax import lax
from jax.experimental import pallas as pl
from jax.experimental.pallas import tpu as pltpu
from jax.experimental.pallas import tpu_sc as plsc

N = 10000
E = 320000
D = 128
HALF = 64
CW = 16
K = 128
NS = 16
G = 16
NSUP = 10
NCHUNK = NSUP * G
EP = NCHUNK * NS * K
NCHUNK_TOT = EP // K
NB = 4
RPT = N // NS
ACCR = N + NS
RPTZ = ACCR // NS


def _make_agg(with_cnt, nb):
    mesh = plsc.VectorSubcoreMesh(core_axis_name="c", subcore_axis_name="s")

    out_type = [jax.ShapeDtypeStruct((N, D), jnp.float32)]
    scratch = [
        pltpu.VMEM_SHARED((ACCR, HALF), jnp.float32),
        pltpu.VMEM_SHARED((N, HALF), jnp.float32),
        pltpu.VMEM((G, 2, K), jnp.int32),
        [pltpu.VMEM((K, HALF), jnp.float32) for _ in range(nb)],
        pltpu.SemaphoreType.DMA((nb,)),
        pltpu.SemaphoreType.DMA((nb,)),
    ]
    if with_cnt:
        out_type.append(jax.ShapeDtypeStruct((2, N, CW), jnp.float32))
        scratch += [
            pltpu.VMEM_SHARED((ACCR, CW), jnp.float32),
            pltpu.VMEM((G // 2, K), jnp.int32),
            pltpu.VMEM((K, CW), jnp.float32),
            pltpu.SemaphoreType.DMA((nb,)),
        ]

    @functools.partial(
        pl.kernel,
        mesh=mesh,
        compiler_params=pltpu.CompilerParams(use_tc_tiling_on_sc=False),
        out_type=out_type,
        scratch_types=scratch,
    )
    def agg(*args):
        if with_cnt:
            (xstk, edges, cdst, zeros, zeros16, ones_hbm, out, outc, acc, xs,
             idx, rows, gsem, ssem, cntacc, cidx, ones_v, csem) = args
        else:
            xstk, edges, zeros, out, acc, xs, idx, rows, gsem, ssem = args
        c = lax.axis_index("c")
        s = lax.axis_index("s")
        pltpu.sync_copy(zeros.at[pl.ds(s * RPTZ, RPTZ)],
                        acc.at[pl.ds(s * RPTZ, RPTZ)])
        pltpu.sync_copy(xstk.at[pl.ds(s * RPT, RPT), pl.ds(c * HALF, HALF)],
                        xs.at[pl.ds(s * RPT, RPT)])
        if with_cnt:
            pltpu.sync_copy(zeros16.at[pl.ds(s * RPTZ, RPTZ)],
                            cntacc.at[pl.ds(s * RPTZ, RPTZ)])
            pltpu.sync_copy(ones_hbm, ones_v)
        plsc.subcore_barrier()

        base = s * NSUP

        def super_body(g, carry):
            pltpu.sync_copy(edges.at[pl.ds((base + g) * G, G)], idx)
            if with_cnt:
                pltpu.sync_copy(
                    cdst.at[c, pl.ds(s * (NSUP * G // 2) + g * (G // 2),
                                     G // 2)], cidx)

            def gather(j):
                b = j % nb
                return pltpu.async_copy(
                    xs.at[idx.at[j, 0]], rows[b], gsem.at[b])

            def wait_cnt(j):
                if with_cnt and j % 2 == 0:
                    cd[j].wait()

            gd = [None] * G
            sd = [None] * G
            cd = [None] * G
            for j in range(nb - 1):
                gd[j] = gather(j)
            for j in range(G):
                b = j % nb
                gd[j].wait()
                sd[j] = pltpu.async_copy(
                    rows[b], acc.at[idx.at[j, 1]], ssem.at[b], add=True)
                if with_cnt and j % 2 == 0:
                    cd[j] = pltpu.async_copy(
                        ones_v, cntacc.at[cidx.at[j // 2]],
                        csem.at[(j // 2) % nb], add=True)
                nxt = j + nb - 1
                if nxt < G:
                    if nxt - nb >= 0:
                        sd[nxt - nb].wait()
                        wait_cnt(nxt - nb)
                    gd[nxt] = gather(nxt)
            for j in range(G - nb, G):
                sd[j].wait()
                wait_cnt(j)
            return carry

        lax.fori_loop(0, NSUP, super_body, 0)
        plsc.subcore_barrier()
        pltpu.sync_copy(acc.at[pl.ds(s * RPT, RPT)],
                        out.at[pl.ds(s * RPT, RPT), pl.ds(c * HALF, HALF)])
        if with_cnt:
            pltpu.sync_copy(cntacc.at[pl.ds(s * RPT, RPT)],
                            outc.at[c, pl.ds(s * RPT, RPT)])

    return agg


_agg_cnt = _make_agg(True, 4)
_agg = _make_agg(False, 5)


BN = 2000


def _tc1_body(p, ct, xr, wl, wr, br, o):
    cnt = jnp.maximum(ct[0, :, :1] + ct[1, :, :1], 1.0)
    mean = p[...] / cnt
    z = (jnp.dot(mean, wl[...], preferred_element_type=jnp.float32)
         + jnp.dot(xr[...], wr[...], preferred_element_type=jnp.float32)
         + br[...])
    o[...] = jnp.maximum(z, 0.0)


def _tc1(sums, cnt, x, Wl1, Wr1, b1):
    return pl.pallas_call(
        _tc1_body,
        grid=(N // BN,),
        in_specs=[
            pl.BlockSpec((BN, D), lambda i: (i, 0)),
            pl.BlockSpec((2, BN, CW), lambda i: (0, i, 0)),
            pl.BlockSpec((BN, D), lambda i: (i, 0)),
            pl.BlockSpec((D, D), lambda i: (0, 0)),
            pl.BlockSpec((D, D), lambda i: (0, 0)),
            pl.BlockSpec((1, D), lambda i: (0, 0)),
        ],
        out_specs=pl.BlockSpec((BN, D), lambda i: (i, 0)),
        out_shape=jax.ShapeDtypeStruct((ACCR, D), jnp.float32),
    )(sums, cnt, x, Wl1, Wr1, b1)


def _tc2_body(a, ct, h2, wl, wr, br, o):
    cnt = jnp.maximum(ct[0, :, :1] + ct[1, :, :1], 1.0)
    mean = a[...] / cnt
    z = (jnp.dot(mean, wl[...], preferred_element_type=jnp.float32)
         + jnp.dot(h2[...], wr[...], preferred_element_type=jnp.float32)
         + br[...])
    o[...] = z


def _tc2(agg2, cnt, h2, Wl2, Wr2, b2):
    return pl.pallas_call(
        _tc2_body,
        grid=(N // BN,),
        in_specs=[
            pl.BlockSpec((BN, D), lambda i: (i, 0)),
            pl.BlockSpec((2, BN, CW), lambda i: (0, i, 0)),
            pl.BlockSpec((BN, D), lambda i: (i, 0)),
            pl.BlockSpec((D, D), lambda i: (0, 0)),
            pl.BlockSpec((D, D), lambda i: (0, 0)),
            pl.BlockSpec((1, D), lambda i: (0, 0)),
        ],
        out_specs=pl.BlockSpec((BN, D), lambda i: (i, 0)),
        out_shape=jax.ShapeDtypeStruct((N, D), jnp.float32),
    )(agg2, cnt, h2, Wl2, Wr2, b2)


def kernel(x, edge_index, Wl1, Wr1, b1, Wl2, Wr2, b2):
    src = edge_index[0].astype(jnp.int32)
    dst = edge_index[1].astype(jnp.int32)
    src_p = jnp.concatenate(
        [src, jnp.zeros((EP - E,), jnp.int32)]).reshape(NCHUNK_TOT, K)
    dst_p = jnp.concatenate(
        [dst, jnp.full((EP - E,), N, jnp.int32)]).reshape(NCHUNK_TOT, K)
    edges = jnp.stack([src_p, dst_p], axis=1)
    cdst = dst_p.reshape(2, NCHUNK_TOT // 2, K)
    zeros = jnp.zeros((ACCR, HALF), jnp.float32)
    zeros16 = jnp.zeros((ACCR, CW), jnp.float32)
    ones16 = jnp.ones((K, CW), jnp.float32)

    sums, cnt = _agg_cnt(x, edges, cdst, zeros, zeros16, ones16)
    h2 = _tc1(sums, cnt, x, Wl1, Wr1, b1.reshape(1, D))
    (agg2,) = _agg(h2, edges, zeros)
    return _tc2(agg2, cnt, h2, Wl2, Wr2, b2.reshape(1, D))


# speedup vs baseline: 10.3999x; 1.0276x over previous
import functools

import jax
import jax.numpy as jnp
from jax import lax
from jax.experimental import pallas as pl
from jax.experimental.pallas import tpu as pltpu
from jax.experimental.pallas import tpu_sc as plsc

N = 10000
E = 320000
D = 128
HALF = 64
CW = 16
K = 128
NS = 16
G = 16
NSUP = 10
NCHUNK = NSUP * G
EP = NCHUNK * NS * K
NCHUNK_TOT = EP // K
NB = 4
RPT = N // NS
ACCR = N + NS
RPTZ = ACCR // NS


def _make_agg(with_cnt, nb, gs):
    mesh = plsc.VectorSubcoreMesh(core_axis_name="c", subcore_axis_name="s")

    out_type = [jax.ShapeDtypeStruct((N, D), jnp.float32)]
    scratch = [
        pltpu.VMEM_SHARED((ACCR, HALF), jnp.float32),
        pltpu.VMEM_SHARED((N, HALF), jnp.float32),
        pltpu.VMEM((gs, 2, K), jnp.int32),
        [pltpu.VMEM((K, HALF), jnp.float32) for _ in range(nb)],
        pltpu.SemaphoreType.DMA((nb,)),
        pltpu.SemaphoreType.DMA((nb,)),
    ]
    if with_cnt:
        out_type.append(jax.ShapeDtypeStruct((2, N, CW), jnp.float32))
        scratch += [
            pltpu.VMEM_SHARED((ACCR, CW), jnp.float32),
            pltpu.VMEM((gs // 2, K), jnp.int32),
            pltpu.VMEM((K, CW), jnp.float32),
            pltpu.SemaphoreType.DMA((nb,)),
        ]

    @functools.partial(
        pl.kernel,
        mesh=mesh,
        compiler_params=pltpu.CompilerParams(use_tc_tiling_on_sc=False),
        out_type=out_type,
        scratch_types=scratch,
    )
    def agg(*args):
        if with_cnt:
            (xstk, edges, cdst, zeros, zeros16, ones_hbm, out, outc, acc, xs,
             idx, rows, gsem, ssem, cntacc, cidx, ones_v, csem) = args
        else:
            xstk, edges, zeros, out, acc, xs, idx, rows, gsem, ssem = args
        c = lax.axis_index("c")
        s = lax.axis_index("s")
        pltpu.sync_copy(zeros.at[pl.ds(s * RPTZ, RPTZ)],
                        acc.at[pl.ds(s * RPTZ, RPTZ)])
        pltpu.sync_copy(xstk.at[pl.ds(s * RPT, RPT), pl.ds(c * HALF, HALF)],
                        xs.at[pl.ds(s * RPT, RPT)])
        if with_cnt:
            pltpu.sync_copy(zeros16.at[pl.ds(s * RPTZ, RPTZ)],
                            cntacc.at[pl.ds(s * RPTZ, RPTZ)])
            pltpu.sync_copy(ones_hbm, ones_v)
        plsc.subcore_barrier()

        nsup = NCHUNK // gs
        base = s * nsup

        def super_body(g, carry):
            pltpu.sync_copy(edges.at[pl.ds((base + g) * gs, gs)], idx)
            if with_cnt:
                pltpu.sync_copy(
                    cdst.at[c, pl.ds(s * (nsup * gs // 2) + g * (gs // 2),
                                     gs // 2)], cidx)

            def gather(j):
                b = j % nb
                return pltpu.async_copy(
                    xs.at[idx.at[j, 0]], rows[b], gsem.at[b])

            def wait_cnt(j):
                if with_cnt and j % 2 == 0:
                    cd[j].wait()

            gd = [None] * gs
            sd = [None] * gs
            cd = [None] * gs
            for j in range(nb - 1):
                gd[j] = gather(j)
            for j in range(gs):
                b = j % nb
                gd[j].wait()
                sd[j] = pltpu.async_copy(
                    rows[b], acc.at[idx.at[j, 1]], ssem.at[b], add=True)
                if with_cnt and j % 2 == 0:
                    cd[j] = pltpu.async_copy(
                        ones_v, cntacc.at[cidx.at[j // 2]],
                        csem.at[(j // 2) % nb], add=True)
                nxt = j + nb - 1
                if nxt < gs:
                    if nxt - nb >= 0:
                        sd[nxt - nb].wait()
                        wait_cnt(nxt - nb)
                    gd[nxt] = gather(nxt)
            for j in range(gs - nb, gs):
                sd[j].wait()
                wait_cnt(j)
            return carry

        lax.fori_loop(0, nsup, super_body, 0)
        plsc.subcore_barrier()
        pltpu.sync_copy(acc.at[pl.ds(s * RPT, RPT)],
                        out.at[pl.ds(s * RPT, RPT), pl.ds(c * HALF, HALF)])
        if with_cnt:
            pltpu.sync_copy(cntacc.at[pl.ds(s * RPT, RPT)],
                            outc.at[c, pl.ds(s * RPT, RPT)])

    return agg


_agg_cnt = _make_agg(True, 4, 16)
_agg = _make_agg(False, 5, 32)


BN = 2000


def _tc1_body(p, ct, xr, wl, wr, br, o):
    cnt = jnp.maximum(ct[0, :, :1] + ct[1, :, :1], 1.0)
    mean = p[...] / cnt
    z = (jnp.dot(mean, wl[...], preferred_element_type=jnp.float32)
         + jnp.dot(xr[...], wr[...], preferred_element_type=jnp.float32)
         + br[...])
    o[...] = jnp.maximum(z, 0.0)


def _tc1(sums, cnt, x, Wl1, Wr1, b1):
    return pl.pallas_call(
        _tc1_body,
        grid=(N // BN,),
        in_specs=[
            pl.BlockSpec((BN, D), lambda i: (i, 0)),
            pl.BlockSpec((2, BN, CW), lambda i: (0, i, 0)),
            pl.BlockSpec((BN, D), lambda i: (i, 0)),
            pl.BlockSpec((D, D), lambda i: (0, 0)),
            pl.BlockSpec((D, D), lambda i: (0, 0)),
            pl.BlockSpec((1, D), lambda i: (0, 0)),
        ],
        out_specs=pl.BlockSpec((BN, D), lambda i: (i, 0)),
        out_shape=jax.ShapeDtypeStruct((ACCR, D), jnp.float32),
    )(sums, cnt, x, Wl1, Wr1, b1)


def _tc2_body(a, ct, h2, wl, wr, br, o):
    cnt = jnp.maximum(ct[0, :, :1] + ct[1, :, :1], 1.0)
    mean = a[...] / cnt
    z = (jnp.dot(mean, wl[...], preferred_element_type=jnp.float32)
         + jnp.dot(h2[...], wr[...], preferred_element_type=jnp.float32)
         + br[...])
    o[...] = z


def _tc2(agg2, cnt, h2, Wl2, Wr2, b2):
    return pl.pallas_call(
        _tc2_body,
        grid=(N // BN,),
        in_specs=[
            pl.BlockSpec((BN, D), lambda i: (i, 0)),
            pl.BlockSpec((2, BN, CW), lambda i: (0, i, 0)),
            pl.BlockSpec((BN, D), lambda i: (i, 0)),
            pl.BlockSpec((D, D), lambda i: (0, 0)),
            pl.BlockSpec((D, D), lambda i: (0, 0)),
            pl.BlockSpec((1, D), lambda i: (0, 0)),
        ],
        out_specs=pl.BlockSpec((BN, D), lambda i: (i, 0)),
        out_shape=jax.ShapeDtypeStruct((N, D), jnp.float32),
    )(agg2, cnt, h2, Wl2, Wr2, b2)


def kernel(x, edge_index, Wl1, Wr1, b1, Wl2, Wr2, b2):
    src = edge_index[0].astype(jnp.int32)
    dst = edge_index[1].astype(jnp.int32)
    src_p = jnp.concatenate(
        [src, jnp.zeros((EP - E,), jnp.int32)]).reshape(NCHUNK_TOT, K)
    dst_p = jnp.concatenate(
        [dst, jnp.full((EP - E,), N, jnp.int32)]).reshape(NCHUNK_TOT, K)
    edges = jnp.stack([src_p, dst_p], axis=1)
    cdst = dst_p.reshape(2, NCHUNK_TOT // 2, K)
    zeros = jnp.zeros((ACCR, HALF), jnp.float32)
    zeros16 = jnp.zeros((ACCR, CW), jnp.float32)
    ones16 = jnp.ones((K, CW), jnp.float32)

    sums, cnt = _agg_cnt(x, edges, cdst, zeros, zeros16, ones16)
    h2 = _tc1(sums, cnt, x, Wl1, Wr1, b1.reshape(1, D))
    (agg2,) = _agg(h2, edges, zeros)
    return _tc2(agg2, cnt, h2, Wl2, Wr2, b2.reshape(1, D))


# speedup vs baseline: 11.4030x; 1.0965x over previous
import functools

import jax
import jax.numpy as jnp
from jax import lax
from jax.experimental import pallas as pl
from jax.experimental.pallas import tpu as pltpu
from jax.experimental.pallas import tpu_sc as plsc

N = 10000
E = 320000
D = 128
HALF = 64
CW = 8
K = 128
NS = 16
G = 16
NSUP = 10
NCHUNK = NSUP * G
EP = NCHUNK * NS * K
NCHUNK_TOT = EP // K
NB = 4
RPT = N // NS
ACCR = N + NS
RPTZ = ACCR // NS


def _make_agg(with_cnt, nb, gs):
    mesh = plsc.VectorSubcoreMesh(core_axis_name="c", subcore_axis_name="s")

    out_type = [jax.ShapeDtypeStruct((N, D), jnp.float32)]
    scratch = [
        pltpu.VMEM_SHARED((ACCR, HALF), jnp.float32),
        pltpu.VMEM_SHARED((N, HALF), jnp.float32),
        pltpu.VMEM((gs, 2, K), jnp.int32),
        [pltpu.VMEM((K, HALF), jnp.float32) for _ in range(nb)],
        pltpu.SemaphoreType.DMA((nb,)),
        pltpu.SemaphoreType.DMA((nb,)),
    ]
    if with_cnt:
        out_type.append(jax.ShapeDtypeStruct((2, N, CW), jnp.float32))
        scratch += [
            pltpu.VMEM_SHARED((ACCR, CW), jnp.float32),
            pltpu.VMEM((gs // 2, K), jnp.int32),
            pltpu.VMEM((K, CW), jnp.float32),
            pltpu.SemaphoreType.DMA((nb,)),
        ]

    @functools.partial(
        pl.kernel,
        mesh=mesh,
        compiler_params=pltpu.CompilerParams(use_tc_tiling_on_sc=False),
        out_type=out_type,
        scratch_types=scratch,
    )
    def agg(*args):
        if with_cnt:
            (xstk, edges, cdst, zeros, zeros16, ones_hbm, out, outc, acc, xs,
             idx, rows, gsem, ssem, cntacc, cidx, ones_v, csem) = args
        else:
            xstk, edges, zeros, out, acc, xs, idx, rows, gsem, ssem = args
        c = lax.axis_index("c")
        s = lax.axis_index("s")
        pltpu.sync_copy(zeros.at[pl.ds(s * RPTZ, RPTZ)],
                        acc.at[pl.ds(s * RPTZ, RPTZ)])
        pltpu.sync_copy(xstk.at[pl.ds(s * RPT, RPT), pl.ds(c * HALF, HALF)],
                        xs.at[pl.ds(s * RPT, RPT)])
        if with_cnt:
            pltpu.sync_copy(zeros16.at[pl.ds(s * RPTZ, RPTZ)],
                            cntacc.at[pl.ds(s * RPTZ, RPTZ)])
            pltpu.sync_copy(ones_hbm, ones_v)
        plsc.subcore_barrier()

        nsup = NCHUNK // gs
        base = s * nsup

        def super_body(g, carry):
            pltpu.sync_copy(edges.at[pl.ds((base + g) * gs, gs)], idx)
            if with_cnt:
                pltpu.sync_copy(
                    cdst.at[c, pl.ds(s * (nsup * gs // 2) + g * (gs // 2),
                                     gs // 2)], cidx)

            def gather(j):
                b = j % nb
                return pltpu.async_copy(
                    xs.at[idx.at[j, 0]], rows[b], gsem.at[b])

            def wait_cnt(j):
                if with_cnt and j % 2 == 0:
                    cd[j].wait()

            gd = [None] * gs
            sd = [None] * gs
            cd = [None] * gs
            for j in range(nb - 1):
                gd[j] = gather(j)
            for j in range(gs):
                b = j % nb
                gd[j].wait()
                sd[j] = pltpu.async_copy(
                    rows[b], acc.at[idx.at[j, 1]], ssem.at[b], add=True)
                if with_cnt and j % 2 == 0:
                    cd[j] = pltpu.async_copy(
                        ones_v, cntacc.at[cidx.at[j // 2]],
                        csem.at[(j // 2) % nb], add=True)
                nxt = j + nb - 1
                if nxt < gs:
                    if nxt - nb >= 0:
                        sd[nxt - nb].wait()
                        wait_cnt(nxt - nb)
                    gd[nxt] = gather(nxt)
            for j in range(gs - nb, gs):
                sd[j].wait()
                wait_cnt(j)
            return carry

        lax.fori_loop(0, nsup, super_body, 0)
        plsc.subcore_barrier()
        pltpu.sync_copy(acc.at[pl.ds(s * RPT, RPT)],
                        out.at[pl.ds(s * RPT, RPT), pl.ds(c * HALF, HALF)])
        if with_cnt:
            pltpu.sync_copy(cntacc.at[pl.ds(s * RPT, RPT)],
                            outc.at[c, pl.ds(s * RPT, RPT)])

    return agg


_agg_cnt = _make_agg(True, 4, 32)
_agg = _make_agg(False, 5, 32)


BN = 2000


def _tc1_body(p, ct, xr, wl, wr, br, o):
    cnt = jnp.maximum(ct[0, :, :1] + ct[1, :, :1], 1.0)
    mean = p[...] / cnt
    z = (jnp.dot(mean, wl[...], preferred_element_type=jnp.float32)
         + jnp.dot(xr[...], wr[...], preferred_element_type=jnp.float32)
         + br[...])
    o[...] = jnp.maximum(z, 0.0)


def _tc1(sums, cnt, x, Wl1, Wr1, b1):
    return pl.pallas_call(
        _tc1_body,
        grid=(N // BN,),
        in_specs=[
            pl.BlockSpec((BN, D), lambda i: (i, 0)),
            pl.BlockSpec((2, BN, CW), lambda i: (0, i, 0)),
            pl.BlockSpec((BN, D), lambda i: (i, 0)),
            pl.BlockSpec((D, D), lambda i: (0, 0)),
            pl.BlockSpec((D, D), lambda i: (0, 0)),
            pl.BlockSpec((1, D), lambda i: (0, 0)),
        ],
        out_specs=pl.BlockSpec((BN, D), lambda i: (i, 0)),
        out_shape=jax.ShapeDtypeStruct((ACCR, D), jnp.float32),
    )(sums, cnt, x, Wl1, Wr1, b1)


def _tc2_body(a, ct, h2, wl, wr, br, o):
    cnt = jnp.maximum(ct[0, :, :1] + ct[1, :, :1], 1.0)
    mean = a[...] / cnt
    z = (jnp.dot(mean, wl[...], preferred_element_type=jnp.float32)
         + jnp.dot(h2[...], wr[...], preferred_element_type=jnp.float32)
         + br[...])
    o[...] = z


def _tc2(agg2, cnt, h2, Wl2, Wr2, b2):
    return pl.pallas_call(
        _tc2_body,
        grid=(N // BN,),
        in_specs=[
            pl.BlockSpec((BN, D), lambda i: (i, 0)),
            pl.BlockSpec((2, BN, CW), lambda i: (0, i, 0)),
            pl.BlockSpec((BN, D), lambda i: (i, 0)),
            pl.BlockSpec((D, D), lambda i: (0, 0)),
            pl.BlockSpec((D, D), lambda i: (0, 0)),
            pl.BlockSpec((1, D), lambda i: (0, 0)),
        ],
        out_specs=pl.BlockSpec((BN, D), lambda i: (i, 0)),
        out_shape=jax.ShapeDtypeStruct((N, D), jnp.float32),
    )(agg2, cnt, h2, Wl2, Wr2, b2)


def kernel(x, edge_index, Wl1, Wr1, b1, Wl2, Wr2, b2):
    src = edge_index[0].astype(jnp.int32)
    dst = edge_index[1].astype(jnp.int32)
    src_p = jnp.concatenate(
        [src, jnp.zeros((EP - E,), jnp.int32)]).reshape(NCHUNK_TOT, K)
    dst_p = jnp.concatenate(
        [dst, jnp.full((EP - E,), N, jnp.int32)]).reshape(NCHUNK_TOT, K)
    edges = jnp.stack([src_p, dst_p], axis=1)
    cdst = dst_p.reshape(2, NCHUNK_TOT // 2, K)
    zeros = jnp.zeros((ACCR, HALF), jnp.float32)
    zeros16 = jnp.zeros((ACCR, CW), jnp.float32)
    ones16 = jnp.ones((K, CW), jnp.float32)

    sums, cnt = _agg_cnt(x, edges, cdst, zeros, zeros16, ones16)
    h2 = _tc1(sums, cnt, x, Wl1, Wr1, b1.reshape(1, D))
    (agg2,) = _agg(h2, edges, zeros)
    return _tc2(agg2, cnt, h2, Wl2, Wr2, b2.reshape(1, D))
